# baseline (speedup 1.0000x reference)
import functools

import jax
import jax.numpy as jnp
from jax import lax
from jax.experimental import pallas as pl
from jax.experimental.pallas import tpu as pltpu

_B, _S, _H = 4, 8192, 2048
_AD, _R, _K = 32, 16, 8
_TEMP = 0.1

_RC = 2048
_CPB = _S // _RC
_N_STEPS = (_B * _S) // _RC


def _tail(pooled, W1, b1, gamma, beta, W2, b2, mask_logits, keep, gumbel,
          training):
    x = jnp.dot(pooled, W1, preferred_element_type=jnp.float32) + b1
    mu = jnp.mean(x, axis=-1, keepdims=True)
    var = jnp.mean((x - mu) ** 2, axis=-1, keepdims=True)
    x = (x - mu) / jnp.sqrt(var + 1e-5) * gamma + beta
    x = 0.5 * x * (1.0 + lax.erf(x / jnp.sqrt(2.0).astype(jnp.float32)))
    x_dropped = jnp.where(keep > 0.5, x / 0.9, 0.0)
    is_training = training != 0
    x = jnp.where(is_training, x_dropped, x)
    logits = (jnp.dot(x, W2, preferred_element_type=jnp.float32) + b2
              + mask_logits)

    def khot(v):
        col = lax.broadcasted_iota(jnp.int32, v.shape, 1)
        cnt = jnp.zeros(v.shape, jnp.int32)
        for k in range(_R):
            vk = v[:, k][:, None]
            beats = (vk > v) | ((vk == v) & (k < col))
            cnt = cnt + beats.astype(jnp.int32)
        return (cnt < _K).astype(jnp.float32)

    noisy = (logits + gumbel) / _TEMP
    hard = khot(noisy)
    z = logits / _TEMP
    z = z - jnp.max(z, axis=-1, keepdims=True)
    ez = jnp.exp(z)
    soft = ez / jnp.sum(ez, axis=-1, keepdims=True)
    mask_train = hard + soft - soft
    mask_eval = khot(logits)
    return jnp.where(is_training, mask_train, mask_eval)


def _fused_kernel(hs_ref, W1_ref, b1_ref, gamma_ref, beta_ref, W2_ref,
                  b2_ref, ml_ref, keep_ref, gumbel_ref, train_ref,
                  out_ref, acc_ref):
    c = pl.program_id(0)
    bb = c // _CPB
    part = jnp.sum(hs_ref[...], axis=0, keepdims=True)

    @pl.when(c % _CPB == 0)
    def _():
        acc_ref[pl.ds(bb, 1), :] = part

    @pl.when(c % _CPB != 0)
    def _():
        acc_ref[pl.ds(bb, 1), :] += part

    @pl.when(c == _N_STEPS - 1)
    def _():
        pooled = acc_ref[...] * (1.0 / _S)
        out_ref[...] = _tail(
            pooled, W1_ref[...], b1_ref[...], gamma_ref[...], beta_ref[...],
            W2_ref[...], b2_ref[...], ml_ref[...], keep_ref[...],
            gumbel_ref[...], train_ref[0, 0])


def kernel(hidden_states, W1, b1, gamma, beta, W2, b2, mask_logits,
           training):
    keep = jax.random.bernoulli(jax.random.key(42), 0.9,
                                (_B, _AD)).astype(jnp.float32)
    u = jax.random.uniform(jax.random.key(7), (_B, _R), dtype=jnp.float32)
    gumbel = -jnp.log(-jnp.log(u + 1e-8) + 1e-8)
    train_arr = jnp.asarray(training, jnp.float32).reshape(1, 1)

    hs2 = hidden_states.reshape(_B * _S, _H)
    tiny = lambda r, c: pl.BlockSpec((r, c), lambda i: (0, 0))
    return pl.pallas_call(
        _fused_kernel,
        grid=(_N_STEPS,),
        in_specs=[
            pl.BlockSpec((_RC, _H), lambda i: (i, 0)),
            tiny(_H, _AD),
            tiny(1, _AD),
            tiny(1, _AD),
            tiny(1, _AD),
            tiny(_AD, _R),
            tiny(1, _R),
            tiny(1, _R),
            tiny(_B, _AD),
            tiny(_B, _R),
            tiny(1, 1),
        ],
        out_specs=pl.BlockSpec((_B, _R), lambda i: (0, 0)),
        out_shape=jax.ShapeDtypeStruct((_B, _R), jnp.float32),
        scratch_shapes=[pltpu.VMEM((_B, _H), jnp.float32)],
    )(hs2, W1, b1.reshape(1, _AD), gamma.reshape(1, _AD),
      beta.reshape(1, _AD), W2, b2.reshape(1, _R),
      mask_logits.reshape(1, _R), keep, gumbel, train_arr)


# speedup vs baseline: 1.0076x; 1.0076x over previous
import functools

import jax
import jax.numpy as jnp
from jax import lax
from jax.experimental import pallas as pl
from jax.experimental.pallas import tpu as pltpu

_B, _S, _H = 4, 8192, 2048
_AD, _R, _K = 32, 16, 8
_TEMP = 0.1

_RC = 1024
_CPB = _S // _RC
_N_STEPS = (_B * _S) // _RC


def _tail(pooled, W1, b1, gamma, beta, W2, b2, mask_logits, keep, gumbel,
          training):
    x = jnp.dot(pooled, W1, preferred_element_type=jnp.float32) + b1
    mu = jnp.mean(x, axis=-1, keepdims=True)
    var = jnp.mean((x - mu) ** 2, axis=-1, keepdims=True)
    x = (x - mu) / jnp.sqrt(var + 1e-5) * gamma + beta
    x = 0.5 * x * (1.0 + lax.erf(x / jnp.sqrt(2.0).astype(jnp.float32)))
    x_dropped = jnp.where(keep > 0.5, x / 0.9, 0.0)
    is_training = training != 0
    x = jnp.where(is_training, x_dropped, x)
    logits = (jnp.dot(x, W2, preferred_element_type=jnp.float32) + b2
              + mask_logits)

    def khot(v):
        col = lax.broadcasted_iota(jnp.int32, v.shape, 1)
        cnt = jnp.zeros(v.shape, jnp.int32)
        for k in range(_R):
            vk = v[:, k][:, None]
            beats = (vk > v) | ((vk == v) & (k < col))
            cnt = cnt + beats.astype(jnp.int32)
        return (cnt < _K).astype(jnp.float32)

    noisy = (logits + gumbel) / _TEMP
    hard = khot(noisy)
    z = logits / _TEMP
    z = z - jnp.max(z, axis=-1, keepdims=True)
    ez = jnp.exp(z)
    soft = ez / jnp.sum(ez, axis=-1, keepdims=True)
    mask_train = hard + soft - soft
    mask_eval = khot(logits)
    return jnp.where(is_training, mask_train, mask_eval)


def _fused_kernel(hs_ref, W1_ref, b1_ref, gamma_ref, beta_ref, W2_ref,
                  b2_ref, ml_ref, keep_ref, gumbel_ref, train_ref,
                  out_ref, acc_ref):
    c = pl.program_id(0)
    bb = c // _CPB

    @pl.when(c == 0)
    def _():
        acc_ref[...] = jnp.zeros_like(acc_ref)

    acc_ref[pl.ds(bb * 8, 8), :] += jnp.sum(hs_ref[...], axis=0)

    @pl.when(c == _N_STEPS - 1)
    def _():
        pooled = jnp.sum(acc_ref[...].reshape(_B, 8, _H), axis=1) * (1.0 / _S)
        out_ref[...] = _tail(
            pooled, W1_ref[...], b1_ref[...], gamma_ref[...], beta_ref[...],
            W2_ref[...], b2_ref[...], ml_ref[...], keep_ref[...],
            gumbel_ref[...], train_ref[0, 0])


def kernel(hidden_states, W1, b1, gamma, beta, W2, b2, mask_logits,
           training):
    keep = jax.random.bernoulli(jax.random.key(42), 0.9,
                                (_B, _AD)).astype(jnp.float32)
    u = jax.random.uniform(jax.random.key(7), (_B, _R), dtype=jnp.float32)
    gumbel = -jnp.log(-jnp.log(u + 1e-8) + 1e-8)
    train_arr = jnp.asarray(training, jnp.float32).reshape(1, 1)

    hs3 = hidden_states.reshape((_B * _S) // 8, 8, _H)
    tiny = lambda r, c: pl.BlockSpec((r, c), lambda i: (0, 0))
    return pl.pallas_call(
        _fused_kernel,
        grid=(_N_STEPS,),
        in_specs=[
            pl.BlockSpec((_RC // 8, 8, _H), lambda i: (i, 0, 0)),
            tiny(_H, _AD),
            tiny(1, _AD),
            tiny(1, _AD),
            tiny(1, _AD),
            tiny(_AD, _R),
            tiny(1, _R),
            tiny(1, _R),
            tiny(_B, _AD),
            tiny(_B, _R),
            tiny(1, 1),
        ],
        out_specs=pl.BlockSpec((_B, _R), lambda i: (0, 0)),
        out_shape=jax.ShapeDtypeStruct((_B, _R), jnp.float32),
        scratch_shapes=[pltpu.VMEM((_B * 8, _H), jnp.float32)],
    )(hs3, W1, b1.reshape(1, _AD), gamma.reshape(1, _AD),
      beta.reshape(1, _AD), W2, b2.reshape(1, _R),
      mask_logits.reshape(1, _R), keep, gumbel, train_arr)
